# baseline (speedup 1.0000x reference)
import functools
import math

import jax
import jax.numpy as jnp
from jax import lax
from jax.experimental import pallas as pl
from jax.experimental.pallas import tpu as pltpu
from jax.experimental.pallas import tpu_sc as plsc

N_ATOMS = 50000
N_EDGES = 800000
NSP = 4
NMAX = 4
LMAX = 3
RC = 5.0
SMOOTH = 0.5
START = RC - SMOOTH

NC = 2
NS = 16
NW = NC * NS

ASUB = 256
SW_PER_W = 7
NSW = NW * SW_PER_W
RSTRIDE = 65
NROWS = ASUB * NSP
ROWS = NROWS * RSTRIDE
DUMP = ROWS
ACC_LEN = ROWS + 128

OUT_LEN = N_ATOMS * NSP * 64
FULL_W = N_ATOMS // ASUB
REM_LEN = (N_ATOMS - FULL_W * ASUB) * NSP * 64

LS = 61
CHUNK = 16 * LS
EMAX = N_EDGES - CHUNK
NZP = 3200
NB = 256

S3 = -1.0 / 6.0
S5 = 1.0 / 120.0
S7 = -1.0 / 5040.0
S9 = 1.0 / 362880.0
S11 = -1.0 / 39916800.0

_PI4 = 4.0 * math.pi
C0 = 0.5 * math.sqrt(1.0 / math.pi)
C1 = math.sqrt(3.0 / _PI4)
C4 = math.sqrt(15.0 / _PI4)
C6 = math.sqrt(5.0 / (16.0 * math.pi))
C8 = math.sqrt(15.0 / (16.0 * math.pi))
C9 = math.sqrt(35.0 / (32.0 * math.pi))
C10 = math.sqrt(105.0 / _PI4)
C11 = math.sqrt(21.0 / (32.0 * math.pi))
C12 = math.sqrt(7.0 / (16.0 * math.pi))
C14 = math.sqrt(105.0 / (16.0 * math.pi))

MU1 = RC / 3.0
MU2 = 2.0 * RC / 3.0


def _sc_body(r_hbm, dv_hbm, ii_hbm, jj_hbm, zp_hbm, bnd_hbm,
             out_hbm, acc, rbuf, dvbuf, iibuf, jjbuf, zpbuf, bbuf, sem):
  cid = lax.axis_index("c")
  sid = lax.axis_index("s")
  wid = sid * NC + cid

  pltpu.sync_copy(zp_hbm, zpbuf)
  pltpu.sync_copy(bnd_hbm, bbuf)

  lstride = lax.iota(jnp.int32, 16) * LS
  zero16 = jnp.zeros((16,), jnp.float32)

  def window_body(s, _):
    swid = wid + NW * s
    base_atom = swid * ASUB
    bwin = bbuf[pl.ds(swid, 16)]
    estart = bwin[0]
    eend = bwin[1]

    def zero_body(i, _):
      for k in range(8):
        acc[pl.ds(i * 128 + k * 16, 16)] = zero16
      return 0

    lax.fori_loop(0, ACC_LEN // 128, zero_body, 0)

    cstart = lax.bitwise_and(estart, jnp.int32(-8))
    total = eend - cstart
    nch = lax.div(total + jnp.int32(CHUNK - 1), jnp.int32(CHUNK))

    def chunk_body(k, _):
      roff = cstart + k * CHUNK
      off = pl.multiple_of(jnp.minimum(roff, EMAX), 8)
      lo = jnp.maximum(estart, roff)
      hi = jnp.minimum(eend, roff + CHUNK)
      c0 = pltpu.async_copy(r_hbm.at[pl.ds(off, CHUNK)], rbuf, sem)
      c1 = pltpu.async_copy(
          dv_hbm.at[pl.ds(pl.multiple_of(off * 3, 8), 3 * CHUNK)], dvbuf, sem)
      c2 = pltpu.async_copy(ii_hbm.at[pl.ds(off, CHUNK)], iibuf, sem)
      c3 = pltpu.async_copy(jj_hbm.at[pl.ds(off, CHUNK)], jjbuf, sem)
      c0.wait(); c1.wait(); c2.wait(); c3.wait()

      def group_body(g, _):
        ev = g + lstride
        ev3 = ev * 3
        rv = plsc.load_gather(rbuf, [ev])
        xv = plsc.load_gather(dvbuf, [ev3])
        yv = plsc.load_gather(dvbuf, [ev3 + 1])
        zv = plsc.load_gather(dvbuf, [ev3 + 2])
        iiv = plsc.load_gather(iibuf, [ev])
        jjv = plsc.load_gather(jjbuf, [ev])

        widx = lax.shift_right_logical(jjv, 4)
        word = plsc.load_gather(zpbuf, [widx])
        shift = lax.shift_left(lax.bitwise_and(jjv, 15), 1)
        sp = lax.bitwise_and(lax.shift_right_logical(word, shift), 3)

        gi = off + ev
        valid = lax.bitwise_and(gi >= lo, gi < hi)
        rowb = ((iiv - base_atom) * NSP + sp) * RSTRIDE
        rowb = jnp.where(valid, rowb, DUMP)

        t = jnp.clip((rv - START) * (1.0 / SMOOTH), 0.0, 1.0)
        u = (t - 0.5) * math.pi
        u2 = u * u
        sinu = u * (1.0 + u2 * (S3 + u2 * (S5 + u2 * (S7 + u2 * (S9 + u2 * S11)))))
        mid = 0.5 - 0.5 * sinu
        fc = jnp.where(rv < START, 1.0, jnp.where(rv < RC, mid, 0.0))

        q = jnp.maximum(rv * (1.0 / RC), 1e-6)
        w0 = fc
        w1 = fc * q
        w2 = w1 * q
        w3 = w2 * q
        d1 = rv - MU1
        d2 = rv - MU2
        d3 = rv - RC
        e0 = jnp.exp(-2.0 * (rv * rv))
        e1 = jnp.exp(-2.0 * (d1 * d1))
        e2 = jnp.exp(-2.0 * (d2 * d2))
        e3 = jnp.exp(-2.0 * (d3 * d3))

        x2 = xv * xv
        y2 = yv * yv
        z2 = zv * zv
        xy = xv * yv
        a0 = w0 * C0
        a1 = (w1 * C1) * yv
        a2 = (w1 * C1) * zv
        a3 = (w1 * C1) * xv
        a4 = (w2 * C4) * xy
        a5 = (w2 * C4) * (yv * zv)
        a6 = (w2 * C6) * (3.0 * z2 - 1.0)
        a7 = (w2 * C4) * (xv * zv)
        a8 = (w2 * C8) * (x2 - y2)
        a9 = (w3 * C9) * (yv * (3.0 * x2 - y2))
        a10 = (w3 * C10) * (xy * zv)
        a11 = (w3 * C11) * (yv * (5.0 * z2 - 1.0))
        a12 = (w3 * C12) * ((5.0 * z2 - 3.0) * zv)
        a13 = (w3 * C11) * (xv * (5.0 * z2 - 1.0))
        a14 = (w3 * C14) * (zv * (x2 - y2))
        a15 = (w3 * C9) * (xv * (x2 - 3.0 * y2))
        am = (a0, a1, a2, a3, a4, a5, a6, a7,
              a8, a9, a10, a11, a12, a13, a14, a15)
        en = (e0, e1, e2, e3)

        for n in range(NMAX):
          for m in range(16):
            plsc.addupdate_scatter(acc, [rowb + (n * 16 + m)], en[n] * am[m])
        return 0

      lax.fori_loop(0, LS, group_body, 0)
      return 0

    lax.fori_loop(0, nch, chunk_body, 0)

    def compact_body(rr, _):
      src = rr * RSTRIDE
      dst = rr * 64
      for k in range(4):
        acc[pl.ds(dst + 16 * k, 16)] = acc[pl.ds(src + 16 * k, 16)]
      return 0

    lax.fori_loop(0, NROWS, compact_body, 0)

    @pl.when(swid < FULL_W)
    def _():
      pltpu.sync_copy(acc.at[pl.ds(0, NROWS * 64)],
                      out_hbm.at[pl.ds(swid * (NROWS * 64), NROWS * 64)])

    @pl.when(swid == FULL_W)
    def _():
      pltpu.sync_copy(acc.at[pl.ds(0, REM_LEN)],
                      out_hbm.at[pl.ds(FULL_W * (NROWS * 64), REM_LEN)])

    return 0

  lax.fori_loop(0, SW_PER_W, window_body, 0)


@jax.jit
def _run(r, dv, ii, jj, zp, bnd):
  mesh = plsc.VectorSubcoreMesh(core_axis_name="c", subcore_axis_name="s",
                                num_cores=NC, num_subcores=NS)
  f = functools.partial(
      pl.kernel, mesh=mesh,
      compiler_params=pltpu.CompilerParams(needs_layout_passes=False),
      out_type=jax.ShapeDtypeStruct((OUT_LEN,), jnp.float32),
      scratch_types=[
          pltpu.VMEM((ACC_LEN,), jnp.float32),
          pltpu.VMEM((CHUNK,), jnp.float32),
          pltpu.VMEM((3 * CHUNK,), jnp.float32),
          pltpu.VMEM((CHUNK,), jnp.int32),
          pltpu.VMEM((CHUNK,), jnp.int32),
          pltpu.VMEM((NZP,), jnp.int32),
          pltpu.VMEM((NB,), jnp.int32),
          pltpu.SemaphoreType.DMA,
      ],
  )(_sc_body)
  return f(r, dv, ii, jj, zp, bnd)


def kernel(distances, direction_vectors, idx_i, idx_j, z):
  r = distances.reshape(N_EDGES)
  dv = direction_vectors.reshape(N_EDGES * 3)
  ii = idx_i.astype(jnp.int32)
  jj = idx_j.astype(jnp.int32)

  spz = ((z >= 6).astype(jnp.int32) + (z >= 7).astype(jnp.int32)
         + (z >= 8).astype(jnp.int32))
  spz = spz.reshape(N_ATOMS // 16, 16)
  shifts = jnp.arange(16, dtype=jnp.int32) * 2
  zp = jnp.sum(spz << shifts[None, :], axis=1).astype(jnp.int32)
  zp = jnp.concatenate([zp, jnp.zeros((NZP - zp.shape[0],), jnp.int32)])

  targets = jnp.arange(NSW + 1, dtype=jnp.int32) * ASUB
  bnd = jnp.searchsorted(idx_i, targets).astype(jnp.int32)
  bnd = jnp.concatenate([bnd, jnp.full((NB - NSW - 1,), N_EDGES, jnp.int32)])

  out = _run(r, dv, ii, jj, zp, bnd)
  return out.reshape(N_ATOMS, NSP, NMAX, (LMAX + 1) ** 2)


# speedup vs baseline: 2.7956x; 2.7956x over previous
import functools
import math

import jax
import jax.numpy as jnp
from jax import lax
from jax.experimental import pallas as pl
from jax.experimental.pallas import tpu as pltpu
from jax.experimental.pallas import tpu_sc as plsc

N_ATOMS = 50000
N_EDGES = 800000
NSP = 4
NMAX = 4
LMAX = 3
RC = 5.0
SMOOTH = 0.5
START = RC - SMOOTH

NC = 2
NS = 16
NW = NC * NS

ASUB = 256
SW_PER_W = 7
NSW = NW * SW_PER_W
RSTRIDE = 65
NROWS = ASUB * NSP
ROWS = NROWS * RSTRIDE
DUMP = ROWS
ACC_LEN = ROWS + 128

OUT_LEN = N_ATOMS * NSP * 64
FULL_W = N_ATOMS // ASUB
REM_LEN = (N_ATOMS - FULL_W * ASUB) * NSP * 64

LS = 61
CHUNK = 16 * LS
EMAX = N_EDGES - CHUNK
NZP = 3200
NB = 256

S3 = -1.0 / 6.0
S5 = 1.0 / 120.0
S7 = -1.0 / 5040.0
S9 = 1.0 / 362880.0
S11 = -1.0 / 39916800.0

_PI4 = 4.0 * math.pi
C0 = 0.5 * math.sqrt(1.0 / math.pi)
C1 = math.sqrt(3.0 / _PI4)
C4 = math.sqrt(15.0 / _PI4)
C6 = math.sqrt(5.0 / (16.0 * math.pi))
C8 = math.sqrt(15.0 / (16.0 * math.pi))
C9 = math.sqrt(35.0 / (32.0 * math.pi))
C10 = math.sqrt(105.0 / _PI4)
C11 = math.sqrt(21.0 / (32.0 * math.pi))
C12 = math.sqrt(7.0 / (16.0 * math.pi))
C14 = math.sqrt(105.0 / (16.0 * math.pi))

MU1 = RC / 3.0
MU2 = 2.0 * RC / 3.0


def _sc_body(r_hbm, x_hbm, y_hbm, zd_hbm, ii_hbm, jj_hbm, zp_hbm, bnd_hbm,
             out_hbm, acc, rbuf, xbuf, ybuf, zdbuf, iibuf, jjbuf, zpbuf,
             bbuf, sem):
  cid = lax.axis_index("c")
  sid = lax.axis_index("s")
  wid = sid * NC + cid

  pltpu.sync_copy(zp_hbm, zpbuf)
  pltpu.sync_copy(bnd_hbm, bbuf)

  lstride = lax.iota(jnp.int32, 16) * LS
  zero16 = jnp.zeros((16,), jnp.float32)

  def window_body(s, _):
    swid = wid + NW * s
    base_atom = swid * ASUB
    bwin = bbuf[pl.ds(swid, 16)]
    estart = bwin[0]
    eend = bwin[1]

    def zero_body(i, _):
      for k in range(8):
        acc[pl.ds(i * 128 + k * 16, 16)] = zero16
      return 0

    lax.fori_loop(0, ACC_LEN // 128, zero_body, 0)

    cstart = lax.bitwise_and(estart, jnp.int32(-8))
    total = eend - cstart
    nch = lax.div(total + jnp.int32(CHUNK - 1), jnp.int32(CHUNK))

    def chunk_body(k, _):
      roff = cstart + k * CHUNK
      off = pl.multiple_of(jnp.minimum(roff, EMAX), 8)
      lo = jnp.maximum(estart, roff)
      hi = jnp.minimum(eend, roff + CHUNK)
      dsl = pl.ds(off, CHUNK)
      c0 = pltpu.async_copy(r_hbm.at[dsl], rbuf, sem)
      c1 = pltpu.async_copy(x_hbm.at[dsl], xbuf, sem)
      c2 = pltpu.async_copy(y_hbm.at[dsl], ybuf, sem)
      c3 = pltpu.async_copy(zd_hbm.at[dsl], zdbuf, sem)
      c4 = pltpu.async_copy(ii_hbm.at[dsl], iibuf, sem)
      c5 = pltpu.async_copy(jj_hbm.at[dsl], jjbuf, sem)
      c0.wait(); c1.wait(); c2.wait(); c3.wait(); c4.wait(); c5.wait()

      def group_body(g, _):
        ev = g + lstride
        rv = plsc.load_gather(rbuf, [ev])
        xv = plsc.load_gather(xbuf, [ev])
        yv = plsc.load_gather(ybuf, [ev])
        zv = plsc.load_gather(zdbuf, [ev])
        iiv = plsc.load_gather(iibuf, [ev])
        jjv = plsc.load_gather(jjbuf, [ev])

        widx = lax.shift_right_logical(jjv, 4)
        word = plsc.load_gather(zpbuf, [widx])
        shift = lax.shift_left(lax.bitwise_and(jjv, 15), 1)
        sp = lax.bitwise_and(lax.shift_right_logical(word, shift), 3)

        gi = off + ev
        valid = lax.bitwise_and(gi >= lo, gi < hi)
        rowb = ((iiv - base_atom) * NSP + sp) * RSTRIDE
        rowb = jnp.where(valid, rowb, DUMP)

        t = jnp.clip((rv - START) * (1.0 / SMOOTH), 0.0, 1.0)
        u = (t - 0.5) * math.pi
        u2 = u * u
        sinu = u * (1.0 + u2 * (S3 + u2 * (S5 + u2 * (S7 + u2 * (S9 + u2 * S11)))))
        mid = 0.5 - 0.5 * sinu
        fc = jnp.where(rv < START, 1.0, jnp.where(rv < RC, mid, 0.0))

        q = jnp.maximum(rv * (1.0 / RC), 1e-6)
        w0 = fc
        w1 = fc * q
        w2 = w1 * q
        w3 = w2 * q
        d1 = rv - MU1
        d2 = rv - MU2
        d3 = rv - RC
        e0 = jnp.exp(-2.0 * (rv * rv))
        e1 = jnp.exp(-2.0 * (d1 * d1))
        e2 = jnp.exp(-2.0 * (d2 * d2))
        e3 = jnp.exp(-2.0 * (d3 * d3))

        x2 = xv * xv
        y2 = yv * yv
        z2 = zv * zv
        xy = xv * yv
        a0 = w0 * C0
        a1 = (w1 * C1) * yv
        a2 = (w1 * C1) * zv
        a3 = (w1 * C1) * xv
        a4 = (w2 * C4) * xy
        a5 = (w2 * C4) * (yv * zv)
        a6 = (w2 * C6) * (3.0 * z2 - 1.0)
        a7 = (w2 * C4) * (xv * zv)
        a8 = (w2 * C8) * (x2 - y2)
        a9 = (w3 * C9) * (yv * (3.0 * x2 - y2))
        a10 = (w3 * C10) * (xy * zv)
        a11 = (w3 * C11) * (yv * (5.0 * z2 - 1.0))
        a12 = (w3 * C12) * ((5.0 * z2 - 3.0) * zv)
        a13 = (w3 * C11) * (xv * (5.0 * z2 - 1.0))
        a14 = (w3 * C14) * (zv * (x2 - y2))
        a15 = (w3 * C9) * (xv * (x2 - 3.0 * y2))
        am = (a0, a1, a2, a3, a4, a5, a6, a7,
              a8, a9, a10, a11, a12, a13, a14, a15)
        en = (e0, e1, e2, e3)

        for n in range(NMAX):
          for m in range(16):
            plsc.addupdate_scatter(acc, [rowb + (n * 16 + m)], en[n] * am[m])
        return 0

      lax.fori_loop(0, LS, group_body, 0)
      return 0

    lax.fori_loop(0, nch, chunk_body, 0)

    def compact_body(rr, _):
      src = rr * RSTRIDE
      dst = rr * 64
      for k in range(4):
        acc[pl.ds(dst + 16 * k, 16)] = acc[pl.ds(src + 16 * k, 16)]
      return 0

    lax.fori_loop(0, NROWS, compact_body, 0)

    @pl.when(swid < FULL_W)
    def _():
      pltpu.sync_copy(acc.at[pl.ds(0, NROWS * 64)],
                      out_hbm.at[pl.ds(swid * (NROWS * 64), NROWS * 64)])

    @pl.when(swid == FULL_W)
    def _():
      pltpu.sync_copy(acc.at[pl.ds(0, REM_LEN)],
                      out_hbm.at[pl.ds(FULL_W * (NROWS * 64), REM_LEN)])

    return 0

  lax.fori_loop(0, SW_PER_W, window_body, 0)


@jax.jit
def _run(r, x, y, zd, ii, jj, zp, bnd):
  mesh = plsc.VectorSubcoreMesh(core_axis_name="c", subcore_axis_name="s",
                                num_cores=NC, num_subcores=NS)
  f = functools.partial(
      pl.kernel, mesh=mesh,
      compiler_params=pltpu.CompilerParams(needs_layout_passes=False),
      out_type=jax.ShapeDtypeStruct((OUT_LEN,), jnp.float32),
      scratch_types=[
          pltpu.VMEM((ACC_LEN,), jnp.float32),
          pltpu.VMEM((CHUNK,), jnp.float32),
          pltpu.VMEM((CHUNK,), jnp.float32),
          pltpu.VMEM((CHUNK,), jnp.float32),
          pltpu.VMEM((CHUNK,), jnp.float32),
          pltpu.VMEM((CHUNK,), jnp.int32),
          pltpu.VMEM((CHUNK,), jnp.int32),
          pltpu.VMEM((NZP,), jnp.int32),
          pltpu.VMEM((NB,), jnp.int32),
          pltpu.SemaphoreType.DMA,
      ],
  )(_sc_body)
  return f(r, x, y, zd, ii, jj, zp, bnd)


def kernel(distances, direction_vectors, idx_i, idx_j, z):
  r = distances[:, 0]
  x = direction_vectors[:, 0]
  y = direction_vectors[:, 1]
  zd = direction_vectors[:, 2]
  ii = idx_i.astype(jnp.int32)
  jj = idx_j.astype(jnp.int32)

  spz = ((z >= 6).astype(jnp.int32) + (z >= 7).astype(jnp.int32)
         + (z >= 8).astype(jnp.int32))
  spz = spz.reshape(N_ATOMS // 16, 16)
  shifts = jnp.arange(16, dtype=jnp.int32) * 2
  zp = jnp.sum(spz << shifts[None, :], axis=1).astype(jnp.int32)
  zp = jnp.concatenate([zp, jnp.zeros((NZP - zp.shape[0],), jnp.int32)])

  targets = jnp.arange(NSW + 1, dtype=jnp.int32) * ASUB
  bnd = jnp.searchsorted(idx_i, targets).astype(jnp.int32)
  bnd = jnp.concatenate([bnd, jnp.full((NB - NSW - 1,), N_EDGES, jnp.int32)])

  out = _run(r, x, y, zd, ii, jj, zp, bnd)
  return out.reshape(N_ATOMS, NSP, NMAX, (LMAX + 1) ** 2)
